# baseline (speedup 1.0000x reference)
import functools

import jax
import jax.numpy as jnp
from jax import lax
from jax.experimental import pallas as pl
from jax.experimental.pallas import tpu as pltpu
from jax.experimental.pallas import tpu_sc as plsc

_N_CAT = 26
_N_NUM = 13
_VOCAB = 100001
_EMBED = 32
_HIDDEN = 64
_BATCH = 16384

_NC = 2
_NS = 16
_NW = _NC * _NS

_BPW = _BATCH // _NW
_CH = 64
_NCHUNK = _BPW // _CH
_ROWS_PER_CHUNK = _CH * _N_CAT
_GPC = _ROWS_PER_CHUNK // 128


def _sc_pooled_embedding(flat_idx2d, flat_table):
    mesh = plsc.VectorSubcoreMesh(core_axis_name="c", subcore_axis_name="s")

    @functools.partial(
        pl.kernel,
        mesh=mesh,
        compiler_params=pltpu.CompilerParams(use_tc_tiling_on_sc=False),
        out_type=jax.ShapeDtypeStruct((_BATCH, _EMBED), jnp.float32),
        scratch_types=[
            pltpu.VMEM((_ROWS_PER_CHUNK,), jnp.int32),
            pltpu.VMEM((_ROWS_PER_CHUNK, _EMBED), jnp.float32),
            pltpu.VMEM((_CH, _EMBED), jnp.float32),
            pltpu.SemaphoreType.DMA,
        ],
    )
    def sc_kernel(idx_hbm, table_hbm, out_hbm, idx_v, rows_v, out_v, sem):
        wid = lax.axis_index("s") * _NC + lax.axis_index("c")

        @pl.loop(0, _NCHUNK)
        def _chunk(k):
            idx_base = (wid * _NCHUNK + k) * _ROWS_PER_CHUNK
            pltpu.sync_copy(idx_hbm.at[pl.ds(idx_base, _ROWS_PER_CHUNK)], idx_v)
            for g in range(_GPC):
                pltpu.async_copy(
                    table_hbm.at[idx_v.at[pl.ds(g * 128, 128)]],
                    rows_v.at[pl.ds(g * 128, 128)],
                    sem,
                )
            pltpu.make_async_copy(
                table_hbm.at[pl.ds(0, _ROWS_PER_CHUNK)], rows_v, sem
            ).wait()

            @pl.loop(0, _CH)
            def _reduce(b):
                base = b * _N_CAT
                for h in range(_EMBED // 16):
                    vals = [
                        rows_v[base + c, pl.ds(16 * h, 16)]
                        for c in range(_N_CAT)
                    ]
                    while len(vals) > 1:
                        vals = [
                            vals[i] + vals[i + 1]
                            if i + 1 < len(vals)
                            else vals[i]
                            for i in range(0, len(vals), 2)
                        ]
                    out_v[b, pl.ds(16 * h, 16)] = vals[0]

            b_base = wid * _BPW + k * _CH
            pltpu.sync_copy(out_v, out_hbm.at[pl.ds(b_base, _CH)])

    return sc_kernel(flat_idx2d, flat_table)


def _dense_body(obj_ref, num_ref, w1t_ref, w2t_ref, out_ref):
    h = obj_ref[...] + jnp.dot(
        num_ref[...], w1t_ref[...], preferred_element_type=jnp.float32
    )
    out_ref[...] = jnp.dot(h, w2t_ref[...], preferred_element_type=jnp.float32)


def _tc_dense(pooled, num, w1t, w2t):
    blk = 2048
    grid = _BATCH // blk
    return pl.pallas_call(
        _dense_body,
        grid=(grid,),
        in_specs=[
            pl.BlockSpec((blk, _EMBED), lambda i: (i, 0)),
            pl.BlockSpec((blk, _N_NUM), lambda i: (i, 0)),
            pl.BlockSpec((_N_NUM, _EMBED), lambda i: (0, 0)),
            pl.BlockSpec((_EMBED, _HIDDEN), lambda i: (0, 0)),
        ],
        out_specs=pl.BlockSpec((blk, _HIDDEN), lambda i: (i, 0)),
        out_shape=jax.ShapeDtypeStruct((_BATCH, _HIDDEN), jnp.float32),
    )(pooled, num, w1t, w2t)


@jax.jit
def kernel(edge_feats, tables, W1, W2):
    cat_idx = edge_feats[:, :_N_CAT].astype(jnp.int32)
    flat_idx = cat_idx + (jnp.arange(_N_CAT, dtype=jnp.int32) * _VOCAB)[None, :]
    flat_idx1d = flat_idx.reshape(_BATCH * _N_CAT)
    flat_table = tables.reshape(_N_CAT * _VOCAB, _EMBED)
    pooled = _sc_pooled_embedding(flat_idx1d, flat_table)
    num = edge_feats[:, _N_CAT:]
    return _tc_dense(pooled, num, W1.T, W2.T)


# speedup vs baseline: 2.8630x; 2.8630x over previous
import functools

import jax
import jax.numpy as jnp
from jax import lax
from jax.experimental import pallas as pl
from jax.experimental.pallas import tpu as pltpu
from jax.experimental.pallas import tpu_sc as plsc

_N_CAT = 26
_N_NUM = 13
_VOCAB = 100001
_VPAD = 100016
_EMBED = 32
_HIDDEN = 64
_BATCH = 16384

_NC = 2
_NS = 16
_NW = _NC * _NS

_IDX_CH = 8192
_NCH = _BATCH // _IDX_CH
_LANES = 16


def _sc_pooled_embedding_t(tab_t, idx_t):
    mesh = plsc.VectorSubcoreMesh(core_axis_name="c", subcore_axis_name="s")

    @functools.partial(
        pl.kernel,
        mesh=mesh,
        compiler_params=pltpu.CompilerParams(
            use_tc_tiling_on_sc=False, needs_layout_passes=False
        ),
        out_type=jax.ShapeDtypeStruct((_EMBED, _BATCH), jnp.float32),
        scratch_types=[
            pltpu.VMEM((_VPAD,), jnp.float32),
            pltpu.VMEM((_IDX_CH,), jnp.int32),
            pltpu.VMEM((_BATCH,), jnp.float32),
        ],
    )
    def sc_kernel(tab_hbm, idx_hbm, out_hbm, vocab_v, idx_v, acc_v):
        wid = lax.axis_index("s") * _NC + lax.axis_index("c")
        zeros = jnp.zeros((_LANES,), jnp.float32)

        @pl.loop(0, _BATCH // _LANES, unroll=8)
        def _zero(j):
            acc_v[pl.ds(j * _LANES, _LANES)] = zeros

        @pl.loop(0, _N_CAT)
        def _feature(c):
            pltpu.sync_copy(tab_hbm.at[c, wid], vocab_v.at[pl.ds(0, _VOCAB)])
            for ch in range(_NCH):
                pltpu.sync_copy(idx_hbm.at[c, pl.ds(ch * _IDX_CH, _IDX_CH)], idx_v)

                @pl.loop(0, _IDX_CH // _LANES, unroll=8)
                def _gather(j):
                    ids = idx_v[pl.ds(j * _LANES, _LANES)]
                    vals = plsc.load_gather(vocab_v, [ids])
                    off = ch * _IDX_CH + j * _LANES
                    acc_v[pl.ds(off, _LANES)] = acc_v[pl.ds(off, _LANES)] + vals

        pltpu.sync_copy(acc_v, out_hbm.at[wid])

    return sc_kernel(tab_t, idx_t)


def _dense_body(pt_ref, num_ref, w1t_ref, w2t_ref, out_ref):
    w12 = jnp.dot(w1t_ref[...], w2t_ref[...], preferred_element_type=jnp.float32)
    obj = lax.dot_general(
        pt_ref[...], w2t_ref[...],
        dimension_numbers=(((0,), (0,)), ((), ())),
        preferred_element_type=jnp.float32,
    )
    out_ref[...] = obj + jnp.dot(num_ref[...], w12, preferred_element_type=jnp.float32)


def _tc_dense(pooled_t, num, w1t, w2t):
    blk = 2048
    grid = _BATCH // blk
    return pl.pallas_call(
        _dense_body,
        grid=(grid,),
        in_specs=[
            pl.BlockSpec((_EMBED, blk), lambda i: (0, i)),
            pl.BlockSpec((blk, _N_NUM), lambda i: (i, 0)),
            pl.BlockSpec((_N_NUM, _EMBED), lambda i: (0, 0)),
            pl.BlockSpec((_EMBED, _HIDDEN), lambda i: (0, 0)),
        ],
        out_specs=pl.BlockSpec((blk, _HIDDEN), lambda i: (i, 0)),
        out_shape=jax.ShapeDtypeStruct((_BATCH, _HIDDEN), jnp.float32),
    )(pooled_t, num, w1t, w2t)


@jax.jit
def kernel(edge_feats, tables, W1, W2):
    tab_t = jnp.transpose(tables, (0, 2, 1))
    idx_t = jnp.transpose(edge_feats[:, :_N_CAT].astype(jnp.int32), (1, 0))
    pooled_t = _sc_pooled_embedding_t(tab_t, idx_t)
    num = edge_feats[:, _N_CAT:]
    return _tc_dense(pooled_t, num, W1.T, W2.T)


# speedup vs baseline: 12.1926x; 4.2587x over previous
import functools

import jax
import jax.numpy as jnp
from jax import lax
from jax.experimental import pallas as pl
from jax.experimental.pallas import tpu as pltpu
from jax.experimental.pallas import tpu_sc as plsc

_N_CAT = 26
_N_NUM = 13
_VOCAB = 100001
_EMBED = 32
_HIDDEN = 64
_BATCH = 16384

_NROWS = _N_CAT * _EMBED
_NRG = _NROWS // 8
_NLT = 784
_LCH = 12544
_NLCH = 8

_NC = 2
_NS = 16
_NW = _NC * _NS

_IDX_CH = 8192
_NCH = _BATCH // _IDX_CH
_LANES = 16


def _repack_body(in_ref, out_ref):
    for k in range(_LCH // 128):
        out_ref[0, k] = in_ref[:, 128 * k:128 * (k + 1)]


def _tc_repack(tab2d):
    return pl.pallas_call(
        _repack_body,
        grid=(_NRG, _NLCH),
        in_specs=[pl.BlockSpec((8, _LCH), lambda i, j: (i, j))],
        out_specs=pl.BlockSpec(
            (1, _LCH // 128, 8, 128), lambda i, j: (i, j, 0, 0)
        ),
        out_shape=jax.ShapeDtypeStruct((_NRG, _NLT, 8, 128), jnp.float32),
    )(tab2d)


def _sc_pooled_embedding_t(tab_packed, idx_t):
    mesh = plsc.VectorSubcoreMesh(core_axis_name="c", subcore_axis_name="s")

    @functools.partial(
        pl.kernel,
        mesh=mesh,
        compiler_params=pltpu.CompilerParams(
            use_tc_tiling_on_sc=False, needs_layout_passes=False
        ),
        out_type=jax.ShapeDtypeStruct((_EMBED, _BATCH), jnp.float32),
        scratch_types=[
            pltpu.VMEM((_NLT, 1, 128), jnp.float32),
            pltpu.VMEM((_IDX_CH,), jnp.int32),
            pltpu.VMEM((_BATCH,), jnp.float32),
        ],
    )
    def sc_kernel(tab_hbm, idx_hbm, out_hbm, vocab_v, idx_v, acc_v):
        wid = lax.axis_index("s") * _NC + lax.axis_index("c")
        zeros = jnp.zeros((_LANES,), jnp.float32)

        @pl.loop(0, _BATCH // _LANES, unroll=8)
        def _zero(j):
            acc_v[pl.ds(j * _LANES, _LANES)] = zeros

        @pl.loop(0, _N_CAT)
        def _feature(c):
            row = c * _EMBED + wid
            rg = row // 8
            s = row % 8
            pltpu.sync_copy(tab_hbm.at[rg, :, pl.ds(s, 1), :], vocab_v)
            for ch in range(_NCH):
                pltpu.sync_copy(idx_hbm.at[c, pl.ds(ch * _IDX_CH, _IDX_CH)], idx_v)

                @pl.loop(0, _IDX_CH // _LANES, unroll=8)
                def _gather(j):
                    ids = idx_v[pl.ds(j * _LANES, _LANES)]
                    lb = lax.shift_right_logical(ids, 7)
                    ln = lax.bitwise_and(ids, 127)
                    vals = plsc.load_gather(vocab_v, [lb, lb * 0, ln])
                    off = ch * _IDX_CH + j * _LANES
                    acc_v[pl.ds(off, _LANES)] = acc_v[pl.ds(off, _LANES)] + vals

        pltpu.sync_copy(acc_v, out_hbm.at[wid])

    return sc_kernel(tab_packed, idx_t)


def _dense_body(pt_ref, num_ref, w1t_ref, w2t_ref, out_ref):
    w12 = jnp.dot(w1t_ref[...], w2t_ref[...], preferred_element_type=jnp.float32)
    obj = lax.dot_general(
        pt_ref[...], w2t_ref[...],
        dimension_numbers=(((0,), (0,)), ((), ())),
        preferred_element_type=jnp.float32,
    )
    out_ref[...] = obj + jnp.dot(num_ref[...], w12, preferred_element_type=jnp.float32)


def _tc_dense(pooled_t, num, w1t, w2t):
    blk = 2048
    grid = _BATCH // blk
    return pl.pallas_call(
        _dense_body,
        grid=(grid,),
        in_specs=[
            pl.BlockSpec((_EMBED, blk), lambda i: (0, i)),
            pl.BlockSpec((blk, _N_NUM), lambda i: (i, 0)),
            pl.BlockSpec((_N_NUM, _EMBED), lambda i: (0, 0)),
            pl.BlockSpec((_EMBED, _HIDDEN), lambda i: (0, 0)),
        ],
        out_specs=pl.BlockSpec((blk, _HIDDEN), lambda i: (i, 0)),
        out_shape=jax.ShapeDtypeStruct((_BATCH, _HIDDEN), jnp.float32),
    )(pooled_t, num, w1t, w2t)


@jax.jit
def kernel(edge_feats, tables, W1, W2):
    tab2d = jnp.transpose(tables, (0, 2, 1)).reshape(_NROWS, _VOCAB)
    tab_packed = _tc_repack(tab2d)
    idx_t = jnp.transpose(edge_feats[:, :_N_CAT].astype(jnp.int32), (1, 0))
    pooled_t = _sc_pooled_embedding_t(tab_packed, idx_t)
    num = edge_feats[:, _N_CAT:]
    return _tc_dense(pooled_t, num, W1.T, W2.T)


# speedup vs baseline: 18.2972x; 1.5007x over previous
import functools

import jax
import jax.numpy as jnp
from jax import lax
from jax.experimental import pallas as pl
from jax.experimental.pallas import tpu as pltpu
from jax.experimental.pallas import tpu_sc as plsc

_N_CAT = 26
_N_NUM = 13
_VOCAB = 100001
_EMBED = 32
_HIDDEN = 64
_BATCH = 16384

_NROWS = _N_CAT * _EMBED
_NRG = _NROWS // 8
_NLT = 784
_LCH = 12544
_NLCH = 8

_NC = 2
_NS = 16
_NW = _NC * _NS

_IDX_CH = 8192
_NCH = _BATCH // _IDX_CH
_LANES = 16


def _repack_body(in_ref, out_ref):
    for r in range(2):
        for k in range(_LCH // 128):
            out_ref[r, k] = in_ref[8 * r:8 * (r + 1), 128 * k:128 * (k + 1)]


def _tc_repack(tab2d):
    return pl.pallas_call(
        _repack_body,
        grid=(_NRG // 2, _NLCH),
        in_specs=[pl.BlockSpec((16, _LCH), lambda i, j: (i, j))],
        out_specs=pl.BlockSpec(
            (2, _LCH // 128, 8, 128), lambda i, j: (i, j, 0, 0)
        ),
        out_shape=jax.ShapeDtypeStruct((_NRG, _NLT, 8, 128), jnp.float32),
    )(tab2d)


def _sc_pooled_embedding_t(tab_packed, idx_t):
    mesh = plsc.VectorSubcoreMesh(core_axis_name="c", subcore_axis_name="s")

    @functools.partial(
        pl.kernel,
        mesh=mesh,
        compiler_params=pltpu.CompilerParams(
            use_tc_tiling_on_sc=False, needs_layout_passes=False
        ),
        out_type=jax.ShapeDtypeStruct((_EMBED, _BATCH), jnp.float32),
        scratch_types=[
            pltpu.VMEM((_NLT, 1, 128), jnp.float32),
            pltpu.VMEM((_IDX_CH,), jnp.int32),
            pltpu.VMEM((_BATCH,), jnp.float32),
        ],
    )
    def sc_kernel(tab_hbm, idx_hbm, out_hbm, vocab_v, idx_v, acc_v):
        wid = lax.axis_index("s") * _NC + lax.axis_index("c")
        zeros = jnp.zeros((_LANES,), jnp.float32)
        izeros = jnp.zeros((_LANES,), jnp.int32)

        @pl.loop(0, _BATCH // _LANES, unroll=8)
        def _zero(j):
            acc_v[pl.ds(j * _LANES, _LANES)] = zeros

        @pl.loop(0, _N_CAT)
        def _feature(c):
            row = c * _EMBED + wid
            rg = row // 8
            s = row % 8
            pltpu.sync_copy(tab_hbm.at[rg, :, pl.ds(s, 1), :], vocab_v)
            for ch in range(_NCH):
                pltpu.sync_copy(idx_hbm.at[c, pl.ds(ch * _IDX_CH, _IDX_CH)], idx_v)

                @pl.loop(0, _IDX_CH // _LANES, unroll=16)
                def _gather(j):
                    ids = idx_v[pl.ds(j * _LANES, _LANES)]
                    lb = lax.shift_right_logical(ids, 7)
                    ln = lax.bitwise_and(ids, 127)
                    vals = plsc.load_gather(vocab_v, [lb, izeros, ln])
                    off = ch * _IDX_CH + j * _LANES
                    plsc.addupdate(acc_v.at[pl.ds(off, _LANES)], vals)

        pltpu.sync_copy(acc_v, out_hbm.at[wid])

    return sc_kernel(tab_packed, idx_t)


def _dense_body(pt_ref, num_ref, w1t_ref, w2t_ref, out_ref):
    w12 = jnp.dot(w1t_ref[...], w2t_ref[...], preferred_element_type=jnp.float32)
    obj = lax.dot_general(
        pt_ref[...], w2t_ref[...],
        dimension_numbers=(((0,), (0,)), ((), ())),
        preferred_element_type=jnp.float32,
    )
    out_ref[...] = obj + jnp.dot(num_ref[...], w12, preferred_element_type=jnp.float32)


def _tc_dense(pooled_t, num, w1t, w2t):
    blk = 2048
    grid = _BATCH // blk
    return pl.pallas_call(
        _dense_body,
        grid=(grid,),
        in_specs=[
            pl.BlockSpec((_EMBED, blk), lambda i: (0, i)),
            pl.BlockSpec((blk, _N_NUM), lambda i: (i, 0)),
            pl.BlockSpec((_N_NUM, _EMBED), lambda i: (0, 0)),
            pl.BlockSpec((_EMBED, _HIDDEN), lambda i: (0, 0)),
        ],
        out_specs=pl.BlockSpec((blk, _HIDDEN), lambda i: (i, 0)),
        out_shape=jax.ShapeDtypeStruct((_BATCH, _HIDDEN), jnp.float32),
    )(pooled_t, num, w1t, w2t)


@jax.jit
def kernel(edge_feats, tables, W1, W2):
    tab2d = jnp.transpose(tables, (0, 2, 1)).reshape(_NROWS, _VOCAB)
    tab_packed = _tc_repack(tab2d)
    idx_t = jnp.transpose(edge_feats[:, :_N_CAT].astype(jnp.int32), (1, 0))
    pooled_t = _sc_pooled_embedding_t(tab_packed, idx_t)
    num = edge_feats[:, _N_CAT:]
    return _tc_dense(pooled_t, num, W1.T, W2.T)


# speedup vs baseline: 21.2781x; 1.1629x over previous
import functools

import jax
import jax.numpy as jnp
from jax import lax
from jax.experimental import pallas as pl
from jax.experimental.pallas import tpu as pltpu
from jax.experimental.pallas import tpu_sc as plsc

_N_CAT = 26
_N_NUM = 13
_VOCAB = 100001
_EMBED = 32
_HIDDEN = 64
_BATCH = 16384

_NGROUP = 2
_CPG = _N_CAT // _NGROUP
_GROWS = _CPG * _EMBED
_GRG = _GROWS // 8

_NLT = 784
_LCH = 12544
_NLCH = 8

_NC = 2
_NS = 16
_NW = _NC * _NS

_IDX_CH = 8192
_NCH = _BATCH // _IDX_CH
_LANES = 16


def _repack_body(in_ref, out_ref):
    for r in range(2):
        for k in range(_LCH // 128):
            out_ref[r, k] = in_ref[8 * r:8 * (r + 1), 128 * k:128 * (k + 1)]


def _tc_repack(tab2d, group):
    base = group * (_GRG // 2)
    return pl.pallas_call(
        _repack_body,
        grid=(_GRG // 2, _NLCH),
        in_specs=[pl.BlockSpec((16, _LCH), lambda i, j: (i + base, j))],
        out_specs=pl.BlockSpec(
            (2, _LCH // 128, 8, 128), lambda i, j: (i, j, 0, 0)
        ),
        out_shape=jax.ShapeDtypeStruct((_GRG, _NLT, 8, 128), jnp.float32),
    )(tab2d)


def _sc_pooled_embedding_t(tab_packed, idx_t, group):
    cbase = group * _CPG
    mesh = plsc.VectorSubcoreMesh(core_axis_name="c", subcore_axis_name="s")

    @functools.partial(
        pl.kernel,
        mesh=mesh,
        compiler_params=pltpu.CompilerParams(
            use_tc_tiling_on_sc=False, needs_layout_passes=False
        ),
        out_type=jax.ShapeDtypeStruct((_EMBED, _BATCH), jnp.float32),
        scratch_types=[
            pltpu.VMEM((_NLT, 1, 128), jnp.float32),
            pltpu.VMEM((_IDX_CH,), jnp.int32),
            pltpu.VMEM((_BATCH,), jnp.float32),
        ],
    )
    def sc_kernel(tab_hbm, idx_hbm, out_hbm, vocab_v, idx_v, acc_v):
        wid = lax.axis_index("s") * _NC + lax.axis_index("c")
        zeros = jnp.zeros((_LANES,), jnp.float32)
        izeros = jnp.zeros((_LANES,), jnp.int32)

        @pl.loop(0, _BATCH // _LANES, unroll=8)
        def _zero(j):
            acc_v[pl.ds(j * _LANES, _LANES)] = zeros

        @pl.loop(0, _CPG)
        def _feature(c):
            row = c * _EMBED + wid
            rg = row // 8
            s = row % 8
            pltpu.sync_copy(tab_hbm.at[rg, :, pl.ds(s, 1), :], vocab_v)
            for ch in range(_NCH):
                pltpu.sync_copy(
                    idx_hbm.at[cbase + c, pl.ds(ch * _IDX_CH, _IDX_CH)], idx_v
                )

                @pl.loop(0, _IDX_CH // _LANES, unroll=16)
                def _gather(j):
                    ids = idx_v[pl.ds(j * _LANES, _LANES)]
                    lb = lax.shift_right_logical(ids, 7)
                    ln = lax.bitwise_and(ids, 127)
                    vals = plsc.load_gather(vocab_v, [lb, izeros, ln])
                    off = ch * _IDX_CH + j * _LANES
                    plsc.addupdate(acc_v.at[pl.ds(off, _LANES)], vals)

        pltpu.sync_copy(acc_v, out_hbm.at[wid])

    return sc_kernel(tab_packed, idx_t)


def _dense_body(p0_ref, p1_ref, num_ref, w1t_ref, w2t_ref, out_ref):
    w12 = jnp.dot(w1t_ref[...], w2t_ref[...], preferred_element_type=jnp.float32)
    pt = p0_ref[...] + p1_ref[...]
    obj = lax.dot_general(
        pt, w2t_ref[...],
        dimension_numbers=(((0,), (0,)), ((), ())),
        preferred_element_type=jnp.float32,
    )
    out_ref[...] = obj + jnp.dot(num_ref[...], w12, preferred_element_type=jnp.float32)


def _tc_dense(p0, p1, num, w1t, w2t):
    blk = 2048
    grid = _BATCH // blk
    return pl.pallas_call(
        _dense_body,
        grid=(grid,),
        in_specs=[
            pl.BlockSpec((_EMBED, blk), lambda i: (0, i)),
            pl.BlockSpec((_EMBED, blk), lambda i: (0, i)),
            pl.BlockSpec((blk, _N_NUM), lambda i: (i, 0)),
            pl.BlockSpec((_N_NUM, _EMBED), lambda i: (0, 0)),
            pl.BlockSpec((_EMBED, _HIDDEN), lambda i: (0, 0)),
        ],
        out_specs=pl.BlockSpec((blk, _HIDDEN), lambda i: (i, 0)),
        out_shape=jax.ShapeDtypeStruct((_BATCH, _HIDDEN), jnp.float32),
    )(p0, p1, num, w1t, w2t)


@jax.jit
def kernel(edge_feats, tables, W1, W2):
    tab2d = jnp.transpose(tables, (0, 2, 1)).reshape(_N_CAT * _EMBED, _VOCAB)
    idx_t = jnp.transpose(edge_feats[:, :_N_CAT].astype(jnp.int32), (1, 0))
    pooled = []
    for g in range(_NGROUP):
        packed_g = _tc_repack(tab2d, g)
        pooled.append(_sc_pooled_embedding_t(packed_g, idx_t, g))
    num = edge_feats[:, _N_CAT:]
    return _tc_dense(pooled[0], pooled[1], num, W1.T, W2.T)


# speedup vs baseline: 22.4460x; 1.0549x over previous
import functools

import jax
import jax.numpy as jnp
from jax import lax
from jax.experimental import pallas as pl
from jax.experimental.pallas import tpu as pltpu
from jax.experimental.pallas import tpu_sc as plsc

_N_CAT = 26
_N_NUM = 13
_VOCAB = 100001
_EMBED = 32
_HIDDEN = 64
_BATCH = 16384

_NGROUP = 2
_CPG = _N_CAT // _NGROUP
_GROWS = _CPG * _EMBED
_GRG = _GROWS // 8

_NLT = 784
_LCH = 12544
_NLCH = 8

_NC = 2
_NS = 16
_NW = _NC * _NS

_IDX_CH = 8192
_NCH = _BATCH // _IDX_CH
_LANES = 16


def _repack_body(in_ref, out_ref):
    for r in range(2):
        for k in range(_LCH // 128):
            out_ref[r, k] = in_ref[8 * r:8 * (r + 1), 128 * k:128 * (k + 1)]


def _tc_repack(tab2d, group):
    base = group * (_GRG // 2)
    return pl.pallas_call(
        _repack_body,
        grid=(_GRG // 2, _NLCH),
        in_specs=[pl.BlockSpec((16, _LCH), lambda i, j: (i + base, j))],
        out_specs=pl.BlockSpec(
            (2, _LCH // 128, 8, 128), lambda i, j: (i, j, 0, 0)
        ),
        out_shape=jax.ShapeDtypeStruct((_GRG, _NLT, 8, 128), jnp.float32),
    )(tab2d)


def _sc_pooled_embedding_t(tab_packed, idx_t, group):
    cbase = group * _CPG
    mesh = plsc.VectorSubcoreMesh(core_axis_name="c", subcore_axis_name="s")

    @functools.partial(
        pl.kernel,
        mesh=mesh,
        compiler_params=pltpu.CompilerParams(
            use_tc_tiling_on_sc=False, needs_layout_passes=False
        ),
        out_type=jax.ShapeDtypeStruct((_EMBED, _BATCH), jnp.float32),
        scratch_types=[
            pltpu.VMEM((_NLT, 1, 128), jnp.float32),
            pltpu.VMEM((_IDX_CH,), jnp.int32),
            pltpu.VMEM((_BATCH,), jnp.float32),
        ],
    )
    def sc_kernel(tab_hbm, idx_hbm, out_hbm, vocab_v, idx_v, acc_v):
        wid = lax.axis_index("s") * _NC + lax.axis_index("c")
        zeros = jnp.zeros((_LANES,), jnp.float32)
        izeros = jnp.zeros((_LANES,), jnp.int32)

        @pl.loop(0, _BATCH // _LANES, unroll=8)
        def _zero(j):
            acc_v[pl.ds(j * _LANES, _LANES)] = zeros

        @pl.loop(0, _CPG)
        def _feature(c):
            row = c * _EMBED + wid
            rg = row // 8
            s = row % 8
            pltpu.sync_copy(tab_hbm.at[rg, :, pl.ds(s, 1), :], vocab_v)
            for ch in range(_NCH):
                pltpu.sync_copy(
                    idx_hbm.at[cbase + c, pl.ds(ch * _IDX_CH, _IDX_CH)], idx_v
                )

                @plsc.parallel_loop(0, _IDX_CH // _LANES, unroll=16)
                def _gather(j):
                    ids = idx_v[pl.ds(j * _LANES, _LANES)]
                    lb = lax.shift_right_logical(ids, 7)
                    ln = lax.bitwise_and(ids, 127)
                    vals = plsc.load_gather(vocab_v, [lb, izeros, ln])
                    off = ch * _IDX_CH + j * _LANES
                    plsc.addupdate(acc_v.at[pl.ds(off, _LANES)], vals)

        pltpu.sync_copy(acc_v, out_hbm.at[wid])

    return sc_kernel(tab_packed, idx_t)


def _dense_body(p0_ref, p1_ref, num_ref, w1t_ref, w2t_ref, out_ref):
    w12 = jnp.dot(w1t_ref[...], w2t_ref[...], preferred_element_type=jnp.float32)
    pt = p0_ref[...] + p1_ref[...]
    obj = lax.dot_general(
        pt, w2t_ref[...],
        dimension_numbers=(((0,), (0,)), ((), ())),
        preferred_element_type=jnp.float32,
    )
    out_ref[...] = obj + jnp.dot(num_ref[...], w12, preferred_element_type=jnp.float32)


def _tc_dense(p0, p1, num, w1t, w2t):
    blk = 2048
    grid = _BATCH // blk
    return pl.pallas_call(
        _dense_body,
        grid=(grid,),
        in_specs=[
            pl.BlockSpec((_EMBED, blk), lambda i: (0, i)),
            pl.BlockSpec((_EMBED, blk), lambda i: (0, i)),
            pl.BlockSpec((blk, _N_NUM), lambda i: (i, 0)),
            pl.BlockSpec((_N_NUM, _EMBED), lambda i: (0, 0)),
            pl.BlockSpec((_EMBED, _HIDDEN), lambda i: (0, 0)),
        ],
        out_specs=pl.BlockSpec((blk, _HIDDEN), lambda i: (i, 0)),
        out_shape=jax.ShapeDtypeStruct((_BATCH, _HIDDEN), jnp.float32),
    )(p0, p1, num, w1t, w2t)


@jax.jit
def kernel(edge_feats, tables, W1, W2):
    tab2d = jnp.transpose(tables, (0, 2, 1)).reshape(_N_CAT * _EMBED, _VOCAB)
    idx_t = jnp.transpose(edge_feats[:, :_N_CAT].astype(jnp.int32), (1, 0))
    pooled = []
    for g in range(_NGROUP):
        packed_g = _tc_repack(tab2d, g)
        pooled.append(_sc_pooled_embedding_t(packed_g, idx_t, g))
    num = edge_feats[:, _N_CAT:]
    return _tc_dense(pooled[0], pooled[1], num, W1.T, W2.T)


# speedup vs baseline: 23.4162x; 1.0432x over previous
import functools

import jax
import jax.numpy as jnp
from jax import lax
from jax.experimental import pallas as pl
from jax.experimental.pallas import tpu as pltpu
from jax.experimental.pallas import tpu_sc as plsc

_N_CAT = 26
_N_NUM = 13
_VOCAB = 100001
_EMBED = 32
_HIDDEN = 64
_BATCH = 16384

_GROUPS = (7, 7, 6, 6)

_NLT = 784
_LCH = 12544
_NLCH = 8

_NC = 2
_NS = 16
_NW = _NC * _NS

_IDX_CH = 8192
_NCH = _BATCH // _IDX_CH
_LANES = 16


def _repack_body(in_ref, out_ref):
    for r in range(2):
        for k in range(_LCH // 128):
            out_ref[r, k] = in_ref[8 * r:8 * (r + 1), 128 * k:128 * (k + 1)]


def _tc_repack(tab2d, cbase, cpg):
    grg = cpg * _EMBED // 8
    base = cbase * _EMBED // 16
    return pl.pallas_call(
        _repack_body,
        grid=(grg // 2, _NLCH),
        in_specs=[pl.BlockSpec((16, _LCH), lambda i, j: (i + base, j))],
        out_specs=pl.BlockSpec(
            (2, _LCH // 128, 8, 128), lambda i, j: (i, j, 0, 0)
        ),
        out_shape=jax.ShapeDtypeStruct((grg, _NLT, 8, 128), jnp.float32),
    )(tab2d)


def _sc_pooled_embedding_t(tab_packed, idx_t, cbase, cpg):
    mesh = plsc.VectorSubcoreMesh(core_axis_name="c", subcore_axis_name="s")

    @functools.partial(
        pl.kernel,
        mesh=mesh,
        compiler_params=pltpu.CompilerParams(
            use_tc_tiling_on_sc=False, needs_layout_passes=False
        ),
        out_type=jax.ShapeDtypeStruct((_EMBED, _BATCH), jnp.float32),
        scratch_types=[
            pltpu.VMEM((_NLT, 1, 128), jnp.float32),
            pltpu.VMEM((_IDX_CH,), jnp.int32),
            pltpu.VMEM((_BATCH,), jnp.float32),
        ],
    )
    def sc_kernel(tab_hbm, idx_hbm, out_hbm, vocab_v, idx_v, acc_v):
        wid = lax.axis_index("s") * _NC + lax.axis_index("c")
        zeros = jnp.zeros((_LANES,), jnp.float32)
        izeros = jnp.zeros((_LANES,), jnp.int32)

        @pl.loop(0, _BATCH // _LANES, unroll=8)
        def _zero(j):
            acc_v[pl.ds(j * _LANES, _LANES)] = zeros

        @pl.loop(0, cpg)
        def _feature(c):
            row = c * _EMBED + wid
            rg = row // 8
            s = row % 8
            pltpu.sync_copy(tab_hbm.at[rg, :, pl.ds(s, 1), :], vocab_v)
            for ch in range(_NCH):
                pltpu.sync_copy(
                    idx_hbm.at[cbase + c, pl.ds(ch * _IDX_CH, _IDX_CH)], idx_v
                )

                @plsc.parallel_loop(0, _IDX_CH // _LANES, unroll=16)
                def _gather(j):
                    ids = idx_v[pl.ds(j * _LANES, _LANES)]
                    lb = lax.shift_right_logical(ids, 7)
                    ln = lax.bitwise_and(ids, 127)
                    vals = plsc.load_gather(vocab_v, [lb, izeros, ln])
                    off = ch * _IDX_CH + j * _LANES
                    plsc.addupdate(acc_v.at[pl.ds(off, _LANES)], vals)

        pltpu.sync_copy(acc_v, out_hbm.at[wid])

    return sc_kernel(tab_packed, idx_t)


def _dense_body(*refs):
    np_ = len(_GROUPS)
    p_refs = refs[:np_]
    num_ref, w1t_ref, w2t_ref, out_ref = refs[np_:]
    w12 = jnp.dot(w1t_ref[...], w2t_ref[...], preferred_element_type=jnp.float32)
    pt = p_refs[0][...]
    for p in p_refs[1:]:
        pt = pt + p[...]
    obj = lax.dot_general(
        pt, w2t_ref[...],
        dimension_numbers=(((0,), (0,)), ((), ())),
        preferred_element_type=jnp.float32,
    )
    out_ref[...] = obj + jnp.dot(num_ref[...], w12, preferred_element_type=jnp.float32)


def _tc_dense(pooled, num, w1t, w2t):
    blk = 2048
    grid = _BATCH // blk
    return pl.pallas_call(
        _dense_body,
        grid=(grid,),
        in_specs=[pl.BlockSpec((_EMBED, blk), lambda i: (0, i))
                  for _ in pooled]
        + [
            pl.BlockSpec((blk, _N_NUM), lambda i: (i, 0)),
            pl.BlockSpec((_N_NUM, _EMBED), lambda i: (0, 0)),
            pl.BlockSpec((_EMBED, _HIDDEN), lambda i: (0, 0)),
        ],
        out_specs=pl.BlockSpec((blk, _HIDDEN), lambda i: (i, 0)),
        out_shape=jax.ShapeDtypeStruct((_BATCH, _HIDDEN), jnp.float32),
    )(*pooled, num, w1t, w2t)


@jax.jit
def kernel(edge_feats, tables, W1, W2):
    tab2d = jnp.transpose(tables, (0, 2, 1)).reshape(_N_CAT * _EMBED, _VOCAB)
    idx_t = jnp.transpose(edge_feats[:, :_N_CAT].astype(jnp.int32), (1, 0))
    pooled = []
    cbase = 0
    for cpg in _GROUPS:
        packed_g = _tc_repack(tab2d, cbase, cpg)
        pooled.append(_sc_pooled_embedding_t(packed_g, idx_t, cbase, cpg))
        cbase += cpg
    num = edge_feats[:, _N_CAT:]
    return _tc_dense(pooled, num, W1.T, W2.T)


# speedup vs baseline: 26.4740x; 1.1306x over previous
import functools

import jax
import jax.numpy as jnp
from jax import lax
from jax.experimental import pallas as pl
from jax.experimental.pallas import tpu as pltpu
from jax.experimental.pallas import tpu_sc as plsc

_N_CAT = 26
_N_NUM = 13
_VOCAB = 100001
_EMBED = 32
_HIDDEN = 64
_BATCH = 16384

_GROUPS = (7, 7, 6, 6)

_NLT = 784
_LCH = 12544
_NLCH = 8

_NC = 2
_NS = 16
_NW = _NC * _NS

_IDX_CH = 8192
_NCH = _BATCH // _IDX_CH
_LANES = 16


def _repack_body(in_ref, out_ref):
    for r in range(4):
        for k in range(_LCH // 128):
            out_ref[r, k] = in_ref[8 * r:8 * (r + 1), 128 * k:128 * (k + 1)]


def _tc_repack(tab2d, cbase, cpg):
    grg = cpg * _EMBED // 8
    base = cbase * _EMBED // 32
    return pl.pallas_call(
        _repack_body,
        grid=(grg // 4, _NLCH),
        in_specs=[pl.BlockSpec((32, _LCH), lambda i, j: (i + base, j))],
        out_specs=pl.BlockSpec(
            (4, _LCH // 128, 8, 128), lambda i, j: (i, j, 0, 0)
        ),
        out_shape=jax.ShapeDtypeStruct((grg, _NLT, 8, 128), jnp.float32),
    )(tab2d)


def _sc_pooled_embedding_t(tab_packed, idx_t, cbase, cpg):
    mesh = plsc.VectorSubcoreMesh(core_axis_name="c", subcore_axis_name="s")

    @functools.partial(
        pl.kernel,
        mesh=mesh,
        compiler_params=pltpu.CompilerParams(
            use_tc_tiling_on_sc=False, needs_layout_passes=False
        ),
        out_type=jax.ShapeDtypeStruct((_EMBED, _BATCH), jnp.float32),
        scratch_types=[
            pltpu.VMEM((_NLT, 1, 128), jnp.float32),
            pltpu.VMEM((_IDX_CH,), jnp.int32),
            pltpu.VMEM((_BATCH,), jnp.float32),
        ],
    )
    def sc_kernel(tab_hbm, idx_hbm, out_hbm, vocab_v, idx_v, acc_v):
        wid = lax.axis_index("s") * _NC + lax.axis_index("c")
        zeros = jnp.zeros((_LANES,), jnp.float32)
        izeros = jnp.zeros((_LANES,), jnp.int32)

        @pl.loop(0, _BATCH // _LANES, unroll=8)
        def _zero(j):
            acc_v[pl.ds(j * _LANES, _LANES)] = zeros

        @pl.loop(0, cpg)
        def _feature(c):
            row = c * _EMBED + wid
            rg = row // 8
            s = row % 8
            pltpu.sync_copy(tab_hbm.at[rg, :, pl.ds(s, 1), :], vocab_v)
            for ch in range(_NCH):
                pltpu.sync_copy(
                    idx_hbm.at[cbase + c, pl.ds(ch * _IDX_CH, _IDX_CH)], idx_v
                )

                @plsc.parallel_loop(0, _IDX_CH // _LANES, unroll=16)
                def _gather(j):
                    ids = idx_v[pl.ds(j * _LANES, _LANES)]
                    lb = lax.shift_right_logical(ids, 7)
                    ln = lax.bitwise_and(ids, 127)
                    vals = plsc.load_gather(vocab_v, [lb, izeros, ln])
                    off = ch * _IDX_CH + j * _LANES
                    plsc.addupdate(acc_v.at[pl.ds(off, _LANES)], vals)

        pltpu.sync_copy(acc_v, out_hbm.at[wid])

    return sc_kernel(tab_packed, idx_t)


def _dense_body(*refs):
    np_ = len(_GROUPS)
    p_refs = refs[:np_]
    num_ref, w1t_ref, w2t_ref, out_ref = refs[np_:]
    w12 = jnp.dot(w1t_ref[...], w2t_ref[...], preferred_element_type=jnp.float32)
    pt = p_refs[0][...]
    for p in p_refs[1:]:
        pt = pt + p[...]
    obj = lax.dot_general(
        pt, w2t_ref[...],
        dimension_numbers=(((0,), (0,)), ((), ())),
        preferred_element_type=jnp.float32,
    )
    out_ref[...] = obj + jnp.dot(num_ref[...], w12, preferred_element_type=jnp.float32)


def _tc_dense(pooled, num, w1t, w2t):
    blk = 2048
    grid = _BATCH // blk
    return pl.pallas_call(
        _dense_body,
        grid=(grid,),
        in_specs=[pl.BlockSpec((_EMBED, blk), lambda i: (0, i))
                  for _ in pooled]
        + [
            pl.BlockSpec((blk, _N_NUM), lambda i: (i, 0)),
            pl.BlockSpec((_N_NUM, _EMBED), lambda i: (0, 0)),
            pl.BlockSpec((_EMBED, _HIDDEN), lambda i: (0, 0)),
        ],
        out_specs=pl.BlockSpec((blk, _HIDDEN), lambda i: (i, 0)),
        out_shape=jax.ShapeDtypeStruct((_BATCH, _HIDDEN), jnp.float32),
    )(*pooled, num, w1t, w2t)


@jax.jit
def kernel(edge_feats, tables, W1, W2):
    tab2d = jnp.transpose(tables, (0, 2, 1)).reshape(_N_CAT * _EMBED, _VOCAB)
    idx_t = jnp.transpose(edge_feats[:, :_N_CAT].astype(jnp.int32), (1, 0))
    pooled = []
    cbase = 0
    for cpg in _GROUPS:
        packed_g = _tc_repack(tab2d, cbase, cpg)
        pooled.append(_sc_pooled_embedding_t(packed_g, idx_t, cbase, cpg))
        cbase += cpg
    num = edge_feats[:, _N_CAT:]
    return _tc_dense(pooled, num, W1.T, W2.T)


# speedup vs baseline: 26.6265x; 1.0058x over previous
import functools

import jax
import jax.numpy as jnp
from jax import lax
from jax.experimental import pallas as pl
from jax.experimental.pallas import tpu as pltpu
from jax.experimental.pallas import tpu_sc as plsc

_N_CAT = 26
_N_NUM = 13
_VOCAB = 100001
_EMBED = 32
_HIDDEN = 64
_BATCH = 16384

_GROUPS = (7, 7, 6, 6)

_NLT = 784
_LCH = 12544
_NLCH = 8

_NC = 2
_NS = 16
_NW = _NC * _NS

_IDX_CH = 8192
_NCH = _BATCH // _IDX_CH
_LANES = 16


def _make_repack_body(rpb):
    def _repack_body(in_ref, out_ref):
        for r in range(rpb):
            for k in range(_LCH // 128):
                out_ref[r, k] = in_ref[8 * r:8 * (r + 1), 128 * k:128 * (k + 1)]
    return _repack_body


def _tc_repack(tab2d, cbase, cpg):
    grg = cpg * _EMBED // 8
    rpb = 7 if grg % 8 else 8
    nblk = grg // rpb
    base = cbase * _EMBED // (8 * rpb)
    return pl.pallas_call(
        _make_repack_body(rpb),
        grid=(nblk, _NLCH),
        in_specs=[pl.BlockSpec((8 * rpb, _LCH), lambda i, j: (i + base, j))],
        out_specs=pl.BlockSpec(
            (rpb, _LCH // 128, 8, 128), lambda i, j: (i, j, 0, 0)
        ),
        out_shape=jax.ShapeDtypeStruct((grg, _NLT, 8, 128), jnp.float32),
    )(tab2d)


def _sc_pooled_embedding_t(tab_packed, idx_t, cbase, cpg):
    mesh = plsc.VectorSubcoreMesh(core_axis_name="c", subcore_axis_name="s")

    @functools.partial(
        pl.kernel,
        mesh=mesh,
        compiler_params=pltpu.CompilerParams(
            use_tc_tiling_on_sc=False, needs_layout_passes=False
        ),
        out_type=jax.ShapeDtypeStruct((_EMBED, _BATCH), jnp.float32),
        scratch_types=[
            pltpu.VMEM((_NLT, 1, 128), jnp.float32),
            pltpu.VMEM((_IDX_CH,), jnp.int32),
            pltpu.VMEM((_BATCH,), jnp.float32),
        ],
    )
    def sc_kernel(tab_hbm, idx_hbm, out_hbm, vocab_v, idx_v, acc_v):
        wid = lax.axis_index("s") * _NC + lax.axis_index("c")
        zeros = jnp.zeros((_LANES,), jnp.float32)
        izeros = jnp.zeros((_LANES,), jnp.int32)

        @pl.loop(0, _BATCH // _LANES, unroll=8)
        def _zero(j):
            acc_v[pl.ds(j * _LANES, _LANES)] = zeros

        @pl.loop(0, cpg)
        def _feature(c):
            row = c * _EMBED + wid
            rg = row // 8
            s = row % 8
            pltpu.sync_copy(tab_hbm.at[rg, :, pl.ds(s, 1), :], vocab_v)
            for ch in range(_NCH):
                pltpu.sync_copy(
                    idx_hbm.at[cbase + c, pl.ds(ch * _IDX_CH, _IDX_CH)], idx_v
                )

                @plsc.parallel_loop(0, _IDX_CH // _LANES, unroll=16)
                def _gather(j):
                    ids = idx_v[pl.ds(j * _LANES, _LANES)]
                    lb = lax.shift_right_logical(ids, 7)
                    ln = lax.bitwise_and(ids, 127)
                    vals = plsc.load_gather(vocab_v, [lb, izeros, ln])
                    off = ch * _IDX_CH + j * _LANES
                    plsc.addupdate(acc_v.at[pl.ds(off, _LANES)], vals)

        pltpu.sync_copy(acc_v, out_hbm.at[wid])

    return sc_kernel(tab_packed, idx_t)


def _dense_body(*refs):
    np_ = len(_GROUPS)
    p_refs = refs[:np_]
    num_ref, w1t_ref, w2t_ref, out_ref = refs[np_:]
    w12 = jnp.dot(w1t_ref[...], w2t_ref[...], preferred_element_type=jnp.float32)
    pt = p_refs[0][...]
    for p in p_refs[1:]:
        pt = pt + p[...]
    obj = lax.dot_general(
        pt, w2t_ref[...],
        dimension_numbers=(((0,), (0,)), ((), ())),
        preferred_element_type=jnp.float32,
    )
    out_ref[...] = obj + jnp.dot(num_ref[...], w12, preferred_element_type=jnp.float32)


def _tc_dense(pooled, num, w1t, w2t):
    blk = 2048
    grid = _BATCH // blk
    return pl.pallas_call(
        _dense_body,
        grid=(grid,),
        in_specs=[pl.BlockSpec((_EMBED, blk), lambda i: (0, i))
                  for _ in pooled]
        + [
            pl.BlockSpec((blk, _N_NUM), lambda i: (i, 0)),
            pl.BlockSpec((_N_NUM, _EMBED), lambda i: (0, 0)),
            pl.BlockSpec((_EMBED, _HIDDEN), lambda i: (0, 0)),
        ],
        out_specs=pl.BlockSpec((blk, _HIDDEN), lambda i: (i, 0)),
        out_shape=jax.ShapeDtypeStruct((_BATCH, _HIDDEN), jnp.float32),
    )(*pooled, num, w1t, w2t)


@jax.jit
def kernel(edge_feats, tables, W1, W2):
    tab2d = jnp.transpose(tables, (0, 2, 1)).reshape(_N_CAT * _EMBED, _VOCAB)
    idx_t = jnp.transpose(edge_feats[:, :_N_CAT].astype(jnp.int32), (1, 0))
    pooled = []
    cbase = 0
    for cpg in _GROUPS:
        packed_g = _tc_repack(tab2d, cbase, cpg)
        pooled.append(_sc_pooled_embedding_t(packed_g, idx_t, cbase, cpg))
        cbase += cpg
    num = edge_feats[:, _N_CAT:]
    return _tc_dense(pooled, num, W1.T, W2.T)
